# baseline (speedup 1.0000x reference)
import functools

import jax
import jax.numpy as jnp
from jax import lax
from jax.experimental import pallas as pl
from jax.experimental.pallas import tpu as pltpu
from jax.experimental.pallas import tpu_sc as plsc

B, V, K = 16384, 2048, 64
_NC, _NS = 2, 16
_NW = _NC * _NS
_WPW = V // _NW
_LANES = 16
_TB = 1024
_GENERAL_ASP = 0


def _zw_body(zf_hbm, idx_hbm, s_hbm, idx_v, flat_v, zw_v, sbuf_v, sem):
    wid = lax.axis_index("s") * _NC + lax.axis_index("c")
    base = wid * _WPW
    pltpu.sync_copy(idx_hbm.at[pl.ds(base, _WPW)], idx_v)
    for j in range(_WPW // _LANES):
        cols = lax.iota(jnp.int32, _LANES) + base + j * _LANES
        rows = idx_v[pl.ds(j * _LANES, _LANES)]
        flat_v[pl.ds(j * _LANES, _LANES)] = rows * V + cols
    pltpu.async_copy(zf_hbm.at[flat_v], zw_v, sem).wait()

    def _zero(i, _):
        sbuf_v[pl.ds(i * _LANES, _LANES)] = jnp.zeros((_LANES,), jnp.float32)
        return _

    lax.fori_loop(0, _WPW * K // _LANES, _zero, 0)
    for j in range(_WPW // _LANES):
        cloc = lax.iota(jnp.int32, _LANES) + j * _LANES
        rows = idx_v[pl.ds(j * _LANES, _LANES)]
        vals = zw_v[pl.ds(j * _LANES, _LANES)]
        plsc.store_scatter(sbuf_v, [cloc * K + rows], vals)
    pltpu.sync_copy(sbuf_v, s_hbm.at[pl.ds(base * K, _WPW * K)])


@functools.cache
def _zw_gather():
    return pl.kernel(
        _zw_body,
        out_type=jax.ShapeDtypeStruct((V * K,), jnp.float32),
        mesh=plsc.VectorSubcoreMesh(
            core_axis_name="c", subcore_axis_name="s", num_cores=_NC, num_subcores=_NS
        ),
        compiler_params=pltpu.CompilerParams(needs_layout_passes=False),
        scratch_types=[
            pltpu.VMEM((_WPW,), jnp.int32),
            pltpu.VMEM((_WPW,), jnp.int32),
            pltpu.VMEM((_WPW,), jnp.float32),
            pltpu.VMEM((_WPW * K,), jnp.float32),
            pltpu.SemaphoreType.DMA,
        ],
    )


_NSPLIT = 4
_CW = V // _NSPLIT


def _bow_copy(bow_hbm, buf_ref, sems, blk, slot, c):
    return pltpu.make_async_copy(
        bow_hbm.at[pl.ds(blk * _TB, _TB), pl.ds(c * _CW, _CW)],
        buf_ref.at[slot, :, pl.ds(c * _CW, _CW)],
        sems.at[slot, c],
    )


def _q_body(s_ref, bow_hbm, out_ref, buf_ref, sems):
    i = pl.program_id(0)
    nblk = pl.num_programs(0)

    @pl.when(i == 0)
    def _prime():
        for c in range(_NSPLIT):
            _bow_copy(bow_hbm, buf_ref, sems, i, i % 2, c).start()

    @pl.when(i + 1 < nblk)
    def _prefetch():
        for c in range(_NSPLIT):
            _bow_copy(bow_hbm, buf_ref, sems, i + 1, (i + 1) % 2, c).start()

    for c in range(_NSPLIT):
        _bow_copy(bow_hbm, buf_ref, sems, i, i % 2, c).wait()

    q = jnp.dot(
        buf_ref[i % 2], s_ref[...], preferred_element_type=jnp.float32
    )
    total = jnp.sum(q, axis=1, keepdims=True)
    col = lax.broadcasted_iota(jnp.int32, q.shape, 1)
    q = jnp.where((total == 0.0) & (col == _GENERAL_ASP), 1e10, q)
    m = jnp.max(q, axis=1, keepdims=True)
    e = jnp.exp(q - m)
    out_ref[...] = e / jnp.sum(e, axis=1, keepdims=True)


def kernel(bow, z, idx2asp):
    zf = z.reshape(-1)
    s_flat = _zw_gather()(zf, idx2asp)
    s_mat = s_flat.reshape(V, K)
    q = pl.pallas_call(
        _q_body,
        grid=(B // _TB,),
        in_specs=[
            pl.BlockSpec((V, K), lambda i: (0, 0)),
            pl.BlockSpec(memory_space=pl.ANY),
        ],
        out_specs=pl.BlockSpec((_TB, K), lambda i: (i, 0)),
        out_shape=jax.ShapeDtypeStruct((B, K), jnp.float32),
        scratch_shapes=[
            pltpu.VMEM((2, _TB, V), jnp.float32),
            pltpu.SemaphoreType.DMA((2, _NSPLIT)),
        ],
    )(s_mat, bow)
    return q


# speedup vs baseline: 1.0143x; 1.0143x over previous
import functools

import jax
import jax.numpy as jnp
from jax import lax
from jax.experimental import pallas as pl
from jax.experimental.pallas import tpu as pltpu
from jax.experimental.pallas import tpu_sc as plsc

B, V, K = 16384, 2048, 64
_NC, _NS = 2, 16
_NW = _NC * _NS
_WPW = V // _NW
_LANES = 16
_TB = 1024
_GENERAL_ASP = 0


def _zw_body(zf_hbm, idx_hbm, s_hbm, idx_v, flat_v, zw_v, sbuf_v, sem):
    wid = lax.axis_index("s") * _NC + lax.axis_index("c")
    base = wid * _WPW
    pltpu.sync_copy(idx_hbm.at[pl.ds(base, _WPW)], idx_v)
    for j in range(_WPW // _LANES):
        cols = lax.iota(jnp.int32, _LANES) + base + j * _LANES
        rows = idx_v[pl.ds(j * _LANES, _LANES)]
        flat_v[pl.ds(j * _LANES, _LANES)] = rows * V + cols
    pltpu.async_copy(zf_hbm.at[flat_v], zw_v, sem).wait()

    def _zero(i, _):
        sbuf_v[pl.ds(i * _LANES, _LANES)] = jnp.zeros((_LANES,), jnp.float32)
        return _

    lax.fori_loop(0, _WPW * K // _LANES, _zero, 0)
    for j in range(_WPW // _LANES):
        cloc = lax.iota(jnp.int32, _LANES) + j * _LANES
        rows = idx_v[pl.ds(j * _LANES, _LANES)]
        vals = zw_v[pl.ds(j * _LANES, _LANES)]
        plsc.store_scatter(sbuf_v, [cloc * K + rows], vals)
    pltpu.sync_copy(sbuf_v, s_hbm.at[pl.ds(base * K, _WPW * K)])


@functools.cache
def _zw_gather():
    return pl.kernel(
        _zw_body,
        out_type=jax.ShapeDtypeStruct((V * K,), jnp.float32),
        mesh=plsc.VectorSubcoreMesh(
            core_axis_name="c", subcore_axis_name="s", num_cores=_NC, num_subcores=_NS
        ),
        compiler_params=pltpu.CompilerParams(needs_layout_passes=False),
        scratch_types=[
            pltpu.VMEM((_WPW,), jnp.int32),
            pltpu.VMEM((_WPW,), jnp.int32),
            pltpu.VMEM((_WPW,), jnp.float32),
            pltpu.VMEM((_WPW * K,), jnp.float32),
            pltpu.SemaphoreType.DMA,
        ],
    )


_NSPLIT = 8
_CW = V // _NSPLIT
_NBUF = 3


def _bow_copy(bow_hbm, buf_ref, sems, blk, slot, c):
    return pltpu.make_async_copy(
        bow_hbm.at[pl.ds(blk * _TB, _TB), pl.ds(c * _CW, _CW)],
        buf_ref.at[slot, :, pl.ds(c * _CW, _CW)],
        sems.at[slot, c],
    )


def _q_body(s_ref, bow_hbm, out_ref, buf_ref, sems):
    i = pl.program_id(0)
    nblk = pl.num_programs(0)

    @pl.when(i == 0)
    def _prime():
        for b in range(_NBUF - 1):
            for c in range(_NSPLIT):
                _bow_copy(bow_hbm, buf_ref, sems, b, b % _NBUF, c).start()

    @pl.when(i + _NBUF - 1 < nblk)
    def _prefetch():
        blk = i + _NBUF - 1
        for c in range(_NSPLIT):
            _bow_copy(bow_hbm, buf_ref, sems, blk, blk % _NBUF, c).start()

    for c in range(_NSPLIT):
        _bow_copy(bow_hbm, buf_ref, sems, i, i % _NBUF, c).wait()

    q = jnp.dot(
        buf_ref[i % _NBUF], s_ref[...], preferred_element_type=jnp.float32
    )
    total = jnp.sum(q, axis=1, keepdims=True)
    col = lax.broadcasted_iota(jnp.int32, q.shape, 1)
    q = jnp.where((total == 0.0) & (col == _GENERAL_ASP), 1e10, q)
    m = jnp.max(q, axis=1, keepdims=True)
    e = jnp.exp(q - m)
    out_ref[...] = e / jnp.sum(e, axis=1, keepdims=True)


def kernel(bow, z, idx2asp):
    zf = z.reshape(-1)
    s_flat = _zw_gather()(zf, idx2asp)
    s_mat = s_flat.reshape(V, K)
    q = pl.pallas_call(
        _q_body,
        grid=(B // _TB,),
        in_specs=[
            pl.BlockSpec((V, K), lambda i: (0, 0)),
            pl.BlockSpec(memory_space=pl.ANY),
        ],
        out_specs=pl.BlockSpec((_TB, K), lambda i: (i, 0)),
        out_shape=jax.ShapeDtypeStruct((B, K), jnp.float32),
        scratch_shapes=[
            pltpu.VMEM((_NBUF, _TB, V), jnp.float32),
            pltpu.SemaphoreType.DMA((_NBUF, _NSPLIT)),
        ],
    )(s_mat, bow)
    return q
